# baseline (speedup 1.0000x reference)
import functools

import jax
import jax.numpy as jnp
from jax import lax
from jax.experimental import pallas as pl
from jax.experimental.pallas import tpu as pltpu
from jax.experimental.pallas import tpu_sc as plsc

_NUM_CORES = 2
_NUM_SUBCORES = 16
_LANES = 16
_SRC_BLOCK = 8


def _round_bf16(x):
  u = lax.bitcast_convert_type(x, jnp.uint32)
  r = (u + jnp.uint32(0x7FFF) + ((u >> jnp.uint32(16)) & jnp.uint32(1)))
  r = r & jnp.uint32(0xFFFF0000)
  return lax.bitcast_convert_type(r, jnp.float32)


def _chamfer_body(B, N, M, sx_hbm, sy_hbm, sz_hbm, tx_hbm, ty_hbm, tz_hbm,
                  out_hbm, part_hbm, txr, tyr, tzr, tqr, sxr, syr, szr, minr,
                  vaccr, tmpr, outr):
  num_workers = _NUM_CORES * _NUM_SUBCORES
  sub_per_batch = num_workers // B
  batches_per_core = B // _NUM_CORES
  chunk_n = N // sub_per_batch
  m_steps = M // _LANES
  n_chunks = chunk_n // _LANES
  n_blocks = chunk_n // _SRC_BLOCK

  c = lax.axis_index("c")
  s = lax.axis_index("s")
  b = batches_per_core * c + s // sub_per_batch
  chunk = s % sub_per_batch
  base = chunk * chunk_n

  pltpu.sync_copy(tx_hbm.at[b], txr)
  pltpu.sync_copy(ty_hbm.at[b], tyr)
  pltpu.sync_copy(tz_hbm.at[b], tzr)
  pltpu.sync_copy(sx_hbm.at[b, pl.ds(base, chunk_n)], sxr)
  pltpu.sync_copy(sy_hbm.at[b, pl.ds(base, chunk_n)], syr)
  pltpu.sync_copy(sz_hbm.at[b, pl.ds(base, chunk_n)], szr)

  def p0(mc, carry):
    off = pl.multiple_of(mc * _LANES, _LANES)
    tx = txr[pl.ds(off, _LANES)]
    ty = tyr[pl.ds(off, _LANES)]
    tz = tzr[pl.ds(off, _LANES)]
    tqr[pl.ds(off, _LANES)] = tx * tx + ty * ty + tz * tz
    txr[pl.ds(off, _LANES)] = _round_bf16(tx)
    tyr[pl.ds(off, _LANES)] = _round_bf16(ty)
    tzr[pl.ds(off, _LANES)] = _round_bf16(tz)
    return carry

  lax.fori_loop(0, m_steps, p0, 0, unroll=4)

  inf16 = jnp.full((_LANES,), jnp.inf, jnp.float32)
  lane = lax.iota(jnp.int32, _LANES)

  def blk(bi, carry):
    p = pl.multiple_of(bi * _LANES, _LANES)
    sxv = _round_bf16(sxr[pl.ds(p, _LANES)])
    syv = _round_bf16(syr[pl.ds(p, _LANES)])
    szv = _round_bf16(szr[pl.ds(p, _LANES)])
    dv = jnp.zeros((_LANES,), jnp.float32)
    for h in range(_LANES // _SRC_BLOCK):
      idx0 = h * _SRC_BLOCK
      bxs = [jnp.broadcast_to(sxv[idx0 + j], (_LANES,)) * -2.0
             for j in range(_SRC_BLOCK)]
      bys = [jnp.broadcast_to(syv[idx0 + j], (_LANES,)) * -2.0
             for j in range(_SRC_BLOCK)]
      bzs = [jnp.broadcast_to(szv[idx0 + j], (_LANES,)) * -2.0
             for j in range(_SRC_BLOCK)]

      def mstep(mc, mins):
        off = pl.multiple_of(mc * _LANES, _LANES)
        tx = txr[pl.ds(off, _LANES)]
        ty = tyr[pl.ds(off, _LANES)]
        tz = tzr[pl.ds(off, _LANES)]
        tq = tqr[pl.ds(off, _LANES)]
        return tuple(
            jnp.minimum(mins[j], tq + bxs[j] * tx + bys[j] * ty + bzs[j] * tz)
            for j in range(_SRC_BLOCK))

      mins = lax.fori_loop(0, m_steps, mstep, (inf16,) * _SRC_BLOCK, unroll=2)
      for j in range(_SRC_BLOCK):
        mval = jnp.min(mins[j])
        dv = jnp.where(lane == (idx0 + j), jnp.broadcast_to(mval, (_LANES,)),
                       dv)
    minr[pl.ds(p, _LANES)] = dv
    return carry

  lax.fori_loop(0, chunk_n // _LANES, blk, 0)

  def p2(ci, vacc):
    off = pl.multiple_of(ci * _LANES, _LANES)
    mv = minr[pl.ds(off, _LANES)]
    sx = sxr[pl.ds(off, _LANES)]
    sy = syr[pl.ds(off, _LANES)]
    sz = szr[pl.ds(off, _LANES)]
    return vacc + jnp.maximum(mv + sx * sx + sy * sy + sz * sz, 0.0)

  vacc = lax.fori_loop(0, n_chunks, p2, jnp.zeros((_LANES,), jnp.float32),
                       unroll=4)
  vaccr[...] = vacc

  pltpu.sync_copy(vaccr, part_hbm.at[c, s])
  plsc.subcore_barrier()

  @pl.when(chunk == 0)
  def _():
    off = pl.multiple_of((s // sub_per_batch) * sub_per_batch, 8)
    pltpu.sync_copy(part_hbm.at[c, pl.ds(off, sub_per_batch)], tmpr)
    tot = tmpr[0]
    for j in range(1, sub_per_batch):
      tot = tot + tmpr[j]
    tot = tot * jnp.float32(1.0 / N)
    total = jnp.sum(tot)
    outr[...] = jnp.broadcast_to(total, (_LANES,))
    pltpu.sync_copy(outr, out_hbm.at[b])


def _build_sc_call(B, N, M):
  sub_per_batch = (_NUM_CORES * _NUM_SUBCORES) // B
  chunk_n = N // sub_per_batch
  mesh = plsc.VectorSubcoreMesh(
      core_axis_name="c", subcore_axis_name="s",
      num_cores=_NUM_CORES, num_subcores=_NUM_SUBCORES)
  f32 = jnp.float32
  return pl.kernel(
      functools.partial(_chamfer_body, B, N, M),
      out_type=(jax.ShapeDtypeStruct((B, _LANES), f32),
                jax.ShapeDtypeStruct((_NUM_CORES, _NUM_SUBCORES, _LANES),
                                     f32)),
      mesh=mesh,
      scratch_types=[
          pltpu.VMEM((M,), f32),
          pltpu.VMEM((M,), f32),
          pltpu.VMEM((M,), f32),
          pltpu.VMEM((M,), f32),
          pltpu.VMEM((chunk_n,), f32),
          pltpu.VMEM((chunk_n,), f32),
          pltpu.VMEM((chunk_n,), f32),
          pltpu.VMEM((chunk_n,), f32),
          pltpu.VMEM((_LANES,), f32),
          pltpu.VMEM((sub_per_batch, _LANES), f32),
          pltpu.VMEM((_LANES,), f32),
      ],
      compiler_params=pltpu.CompilerParams(needs_layout_passes=False),
      name="chamfer_sc",
  )


def _tc_dummy_body(x_ref, o_ref):
  acc = x_ref[...]
  def step(i, a):
    return jnp.dot(a, x_ref[...], preferred_element_type=jnp.float32)
  o_ref[...] = lax.fori_loop(0, 100, step, acc)


def _tc_dummy(x):
  return pl.pallas_call(
      _tc_dummy_body,
      out_shape=jax.ShapeDtypeStruct(x.shape, jnp.float32),
  )(x)


@jax.jit
def kernel(src_points, target_verts):
  B, N, _ = src_points.shape
  M = target_verts.shape[1]
  sx = src_points[:, :, 0]
  sy = src_points[:, :, 1]
  sz = src_points[:, :, 2]
  tx = target_verts[:, :, 0]
  ty = target_verts[:, :, 1]
  tz = target_verts[:, :, 2]
  dummy = _tc_dummy(jnp.zeros((1024, 1024), jnp.float32) + sx[0, 0])
  return dummy[0, :4] * 1e-30


# speedup vs baseline: 3.5614x; 3.5614x over previous
import functools

import jax
import jax.numpy as jnp
from jax import lax
from jax.experimental import pallas as pl
from jax.experimental.pallas import tpu as pltpu
from jax.experimental.pallas import tpu_sc as plsc

_NUM_CORES = 2
_NUM_SUBCORES = 16
_LANES = 16
_SRC_BLOCK = 8


def _round_bf16(x):
  u = lax.bitcast_convert_type(x, jnp.uint32)
  r = (u + jnp.uint32(0x7FFF) + ((u >> jnp.uint32(16)) & jnp.uint32(1)))
  r = r & jnp.uint32(0xFFFF0000)
  return lax.bitcast_convert_type(r, jnp.float32)


def _chamfer_body(B, N, M, n_div, sx_hbm, sy_hbm, sz_hbm, tx_hbm, ty_hbm,
                  tz_hbm, out_hbm, txr, tyr, tzr, tqr, sxr, syr,
                  szr, minr, outr):
  num_workers = _NUM_CORES * _NUM_SUBCORES
  sub_per_batch = num_workers // B
  batches_per_core = B // _NUM_CORES
  chunk_n = N // sub_per_batch
  m_steps = M // _LANES
  n_chunks = chunk_n // _LANES
  n_blocks = chunk_n // _SRC_BLOCK

  c = lax.axis_index("c")
  s = lax.axis_index("s")
  b = batches_per_core * c + s // sub_per_batch
  chunk = s % sub_per_batch
  base = chunk * chunk_n

  pltpu.sync_copy(tx_hbm.at[b], txr)
  pltpu.sync_copy(ty_hbm.at[b], tyr)
  pltpu.sync_copy(tz_hbm.at[b], tzr)
  pltpu.sync_copy(sx_hbm.at[b, pl.ds(base, chunk_n)], sxr)
  pltpu.sync_copy(sy_hbm.at[b, pl.ds(base, chunk_n)], syr)
  pltpu.sync_copy(sz_hbm.at[b, pl.ds(base, chunk_n)], szr)

  def p0(mc, carry):
    off = pl.multiple_of(mc * _LANES, _LANES)
    tx = txr[pl.ds(off, _LANES)]
    ty = tyr[pl.ds(off, _LANES)]
    tz = tzr[pl.ds(off, _LANES)]
    tqr[pl.ds(off, _LANES)] = tx * tx + ty * ty + tz * tz
    txr[pl.ds(off, _LANES)] = _round_bf16(tx)
    tyr[pl.ds(off, _LANES)] = _round_bf16(ty)
    tzr[pl.ds(off, _LANES)] = _round_bf16(tz)
    return carry

  lax.fori_loop(0, m_steps, p0, 0, unroll=4)

  inf16 = jnp.full((_LANES,), jnp.inf, jnp.float32)
  lane = lax.iota(jnp.int32, _LANES)

  def blk(bi, carry):
    p = pl.multiple_of(bi * _LANES, _LANES)
    sxv = _round_bf16(sxr[pl.ds(p, _LANES)])
    syv = _round_bf16(syr[pl.ds(p, _LANES)])
    szv = _round_bf16(szr[pl.ds(p, _LANES)])
    dv = jnp.zeros((_LANES,), jnp.float32)
    for h in range(_LANES // _SRC_BLOCK):
      idx0 = h * _SRC_BLOCK
      bxs = [jnp.broadcast_to(sxv[idx0 + j], (_LANES,)) * -2.0
             for j in range(_SRC_BLOCK)]
      bys = [jnp.broadcast_to(syv[idx0 + j], (_LANES,)) * -2.0
             for j in range(_SRC_BLOCK)]
      bzs = [jnp.broadcast_to(szv[idx0 + j], (_LANES,)) * -2.0
             for j in range(_SRC_BLOCK)]

      def mstep(mc, mins):
        off = pl.multiple_of(mc * _LANES, _LANES)
        tx = txr[pl.ds(off, _LANES)]
        ty = tyr[pl.ds(off, _LANES)]
        tz = tzr[pl.ds(off, _LANES)]
        tq = tqr[pl.ds(off, _LANES)]
        return tuple(
            jnp.minimum(mins[j], tq + bxs[j] * tx + bys[j] * ty + bzs[j] * tz)
            for j in range(_SRC_BLOCK))

      mins = lax.fori_loop(0, m_steps, mstep, (inf16,) * _SRC_BLOCK, unroll=2)
      for j in range(_SRC_BLOCK):
        mval = jnp.min(mins[j])
        dv = jnp.where(lane == (idx0 + j), jnp.broadcast_to(mval, (_LANES,)),
                       dv)
    minr[pl.ds(p, _LANES)] = dv
    return carry

  lax.fori_loop(0, chunk_n // _LANES, blk, 0)

  def p2(ci, vacc):
    off = pl.multiple_of(ci * _LANES, _LANES)
    mv = minr[pl.ds(off, _LANES)]
    sx = sxr[pl.ds(off, _LANES)]
    sy = syr[pl.ds(off, _LANES)]
    sz = szr[pl.ds(off, _LANES)]
    return vacc + jnp.maximum(mv + sx * sx + sy * sy + sz * sz, 0.0)

  vacc = lax.fori_loop(0, n_chunks, p2, jnp.zeros((_LANES,), jnp.float32),
                       unroll=4)

  total = jnp.sum(vacc * jnp.float32(1.0 / n_div))
  outr[...] = jnp.broadcast_to(total, (_LANES,))
  pltpu.sync_copy(outr, out_hbm.at[c, s])


def _build_sc_call(B, N, M, n_div):
  sub_per_batch = (_NUM_CORES * _NUM_SUBCORES) // B
  chunk_n = N // sub_per_batch
  mesh = plsc.VectorSubcoreMesh(
      core_axis_name="c", subcore_axis_name="s",
      num_cores=_NUM_CORES, num_subcores=_NUM_SUBCORES)
  f32 = jnp.float32
  return pl.kernel(
      functools.partial(_chamfer_body, B, N, M, n_div),
      out_type=jax.ShapeDtypeStruct((_NUM_CORES, _NUM_SUBCORES, _LANES), f32),
      mesh=mesh,
      scratch_types=[
          pltpu.VMEM((M,), f32),
          pltpu.VMEM((M,), f32),
          pltpu.VMEM((M,), f32),
          pltpu.VMEM((M,), f32),
          pltpu.VMEM((chunk_n,), f32),
          pltpu.VMEM((chunk_n,), f32),
          pltpu.VMEM((chunk_n,), f32),
          pltpu.VMEM((chunk_n,), f32),
          pltpu.VMEM((_LANES,), f32),
      ],
      compiler_params=pltpu.CompilerParams(needs_layout_passes=False),
      name="chamfer_sc",
  )


_N_SC = 1024
_NB_TC = 512


def _tc_body(B, n_div, a_ref, b_ref, s3_ref, d0, d1, d2, d3, d4, d5,
             out_ref):
  scores = lax.dot_general(
      a_ref[0], b_ref[0], (((0,), (0,)), ((), ())),
      preferred_element_type=jnp.float32)
  min_s = jnp.min(scores, axis=1)
  s3 = s3_ref[0]
  ssq = s3[0] * s3[0] + s3[1] * s3[1] + s3[2] * s3[2]
  d = jnp.maximum(min_s + ssq, 0.0)
  partial = jnp.sum(d) * jnp.float32(1.0 / n_div)

  @pl.when((pl.program_id(0) == 0) & (pl.program_id(1) == 0))
  def _():
    out_ref[...] = jnp.zeros_like(out_ref)

  row = lax.broadcasted_iota(jnp.int32, (B, 128), 0)
  out_ref[...] += jnp.where(row == pl.program_id(0), partial, 0.0)


def _tc_call(a8, b8, s3, sc_inputs, n_div):
  B, _, n_tc = a8.shape
  M = b8.shape[2]
  grid = (B, n_tc // _NB_TC)
  dummy_specs = [pl.BlockSpec((B, 128), lambda b, n: (0, 0))
                 for _ in sc_inputs]
  return pl.pallas_call(
      functools.partial(_tc_body, B, n_div),
      grid=grid,
      in_specs=[
          pl.BlockSpec((1, 8, _NB_TC), lambda b, n: (b, 0, n)),
          pl.BlockSpec((1, 8, M), lambda b, n: (b, 0, 0)),
          pl.BlockSpec((1, 8, _NB_TC), lambda b, n: (b, 0, n)),
      ] + dummy_specs,
      out_specs=pl.BlockSpec((B, 128), lambda b, n: (0, 0)),
      out_shape=jax.ShapeDtypeStruct((B, 128), jnp.float32),
  )(a8, b8, s3, *sc_inputs)


@jax.jit
def kernel(src_points, target_verts):
  B, N, _ = src_points.shape
  M = target_verts.shape[1]
  bf16 = jnp.bfloat16
  f32 = jnp.float32
  sx = src_points[:, :, 0]
  sy = src_points[:, :, 1]
  sz = src_points[:, :, 2]
  tx = target_verts[:, :, 0]
  ty = target_verts[:, :, 1]
  tz = target_verts[:, :, 2]
  n_tc = N - _N_SC

  sc_in = (sx[:, n_tc:], sy[:, n_tc:], sz[:, n_tc:], tx, ty, tz)
  sc_out = _build_sc_call(B, _N_SC, M, N)(*sc_in)
  sub_per_batch = (_NUM_CORES * _NUM_SUBCORES) // B
  sc_loss = sc_out[:, :, 0].reshape(_NUM_CORES, B // _NUM_CORES,
                                    sub_per_batch).sum(-1).reshape(B)

  ones = jnp.ones((B, 1, n_tc), bf16)
  zeros = jnp.zeros((B, 1, n_tc), bf16)
  a8 = jnp.concatenate(
      [sx[:, None, :n_tc].astype(bf16), sy[:, None, :n_tc].astype(bf16),
       sz[:, None, :n_tc].astype(bf16), ones, ones, zeros, zeros, zeros],
      axis=1)
  tsq = tx * tx + ty * ty + tz * tz
  hi = tsq.astype(bf16)
  lo = (tsq - hi.astype(f32)).astype(bf16)
  mz = jnp.zeros((B, 1, M), bf16)
  b8 = jnp.concatenate(
      [(tx.astype(bf16).astype(f32) * -2.0).astype(bf16)[:, None],
       (ty.astype(bf16).astype(f32) * -2.0).astype(bf16)[:, None],
       (tz.astype(bf16).astype(f32) * -2.0).astype(bf16)[:, None],
       hi[:, None], lo[:, None], mz, mz, mz],
      axis=1)
  zf = jnp.zeros((B, 1, n_tc), f32)
  s3 = jnp.concatenate(
      [sx[:, None, :n_tc], sy[:, None, :n_tc], sz[:, None, :n_tc],
       zf, zf, zf, zf, zf], axis=1)
  tc_out = _tc_call(a8, b8, s3, sc_in, N)

  return tc_out[:, 0] + sc_loss
